# Initial kernel scaffold; baseline (speedup 1.0000x reference)
import functools

import jax
import jax.numpy as jnp
from jax.experimental import pallas as pl
from jax.experimental.pallas import tpu as pltpu
from jax.experimental.pallas import tpu_sc as plsc

N_NODES = 10000
N_EDGES = 320000
HIDDEN = 128
EDGE_DIM = 16
OUT_DIM = 2 * HIDDEN
BN_EPS = 1e-5

NUM_CORES = 2
NUM_SUBCORES = 16
NUM_WORKERS = NUM_CORES * NUM_SUBCORES

GATHER_W = 128
EDGE_TILE = 1280
N_TILES = N_EDGES // EDGE_TILE

CHUNK = 80
EDGES_PER_WORKER = N_EDGES // NUM_WORKERS
ROWS_PER_SUBCORE = N_NODES // NUM_SUBCORES
ZROWS = 125


def _sc_mesh():
    return plsc.VectorSubcoreMesh(core_axis_name="core",
                                  subcore_axis_name="subcore")


def _sc_gather(node_feats, src_idx, dst_idx):
    out_t = jax.ShapeDtypeStruct((N_EDGES, HIDDEN), node_feats.dtype)

    @functools.partial(pl.kernel, out_type=(out_t, out_t), mesh=_sc_mesh())
    def k(nf_hbm, si_hbm, di_hbm, os_hbm, od_hbm):
        def body(si_v, di_v, os_v, od_v):
            pltpu.sync_copy(nf_hbm.at[si_v.at[0]], os_v)
            pltpu.sync_copy(nf_hbm.at[di_v.at[0]], od_v)

        pltpu.emit_pipeline(
            body,
            grid=(N_EDGES // GATHER_W,),
            in_specs=[
                pl.BlockSpec((1, GATHER_W), lambda i: (0, i)),
                pl.BlockSpec((1, GATHER_W), lambda i: (0, i)),
            ],
            out_specs=[
                pl.BlockSpec((GATHER_W, HIDDEN), lambda i: (i, 0)),
                pl.BlockSpec((GATHER_W, HIDDEN), lambda i: (i, 0)),
            ],
            core_axis_name=("core", "subcore"),
            dimension_semantics=(pltpu.PARALLEL,),
        )(si_hbm, di_hbm, os_hbm, od_hbm)

    return k(node_feats, src_idx, dst_idx)


def _pass_a(src_rows, dst_rows, edge_feats, w12t, w3t, b_row):
    def body(src_ref, dst_ref, ef_ref, w12_ref, w3_ref, b_ref,
             z_ref, s1_ref, s2_ref):
        x = jnp.concatenate([src_ref[...], dst_ref[...]], axis=1)
        z = jnp.dot(x.astype(jnp.bfloat16), w12_ref[...],
                    preferred_element_type=jnp.float32)
        z = z + jnp.dot(ef_ref[...].astype(jnp.bfloat16), w3_ref[...],
                        preferred_element_type=jnp.float32)
        z = z + b_ref[...]
        z_ref[...] = z.astype(jnp.bfloat16)

        @pl.when(pl.program_id(0) == 0)
        def _():
            s1_ref[...] = jnp.zeros_like(s1_ref)
            s2_ref[...] = jnp.zeros_like(s2_ref)

        s1_ref[...] += jnp.sum(z, axis=0, keepdims=True)
        s2_ref[...] += jnp.sum(z * z, axis=0, keepdims=True)

    return pl.pallas_call(
        body,
        grid=(N_TILES,),
        in_specs=[
            pl.BlockSpec((EDGE_TILE, HIDDEN), lambda i: (i, 0)),
            pl.BlockSpec((EDGE_TILE, HIDDEN), lambda i: (i, 0)),
            pl.BlockSpec((EDGE_TILE, EDGE_DIM), lambda i: (i, 0)),
            pl.BlockSpec((2 * HIDDEN, OUT_DIM), lambda i: (0, 0)),
            pl.BlockSpec((EDGE_DIM, OUT_DIM), lambda i: (0, 0)),
            pl.BlockSpec((1, OUT_DIM), lambda i: (0, 0)),
        ],
        out_specs=[
            pl.BlockSpec((EDGE_TILE, OUT_DIM), lambda i: (i, 0)),
            pl.BlockSpec((1, OUT_DIM), lambda i: (0, 0)),
            pl.BlockSpec((1, OUT_DIM), lambda i: (0, 0)),
        ],
        out_shape=[
            jax.ShapeDtypeStruct((N_EDGES, OUT_DIM), jnp.bfloat16),
            jax.ShapeDtypeStruct((1, OUT_DIM), jnp.float32),
            jax.ShapeDtypeStruct((1, OUT_DIM), jnp.float32),
        ],
    )(src_rows, dst_rows, edge_feats, w12t, w3t, b_row)


def _pass_b(z_bf, s1, s2, gamma_row, beta_row):
    def body(z_ref, s1_ref, s2_ref, g_ref, be_ref, m_ref):
        inv_n = jnp.float32(1.0 / N_EDGES)
        mean = s1_ref[...] * inv_n
        var = s2_ref[...] * inv_n - mean * mean
        scale = g_ref[...] * jax.lax.rsqrt(var + BN_EPS)
        shift = be_ref[...] - mean * scale
        zn = z_ref[...].astype(jnp.float32) * scale + shift
        gate = zn[:, :HIDDEN]
        msg = zn[:, HIDDEN:]
        m_ref[...] = jax.nn.sigmoid(gate) * jax.nn.softplus(msg)

    return pl.pallas_call(
        body,
        grid=(N_TILES,),
        in_specs=[
            pl.BlockSpec((EDGE_TILE, OUT_DIM), lambda i: (i, 0)),
            pl.BlockSpec((1, OUT_DIM), lambda i: (0, 0)),
            pl.BlockSpec((1, OUT_DIM), lambda i: (0, 0)),
            pl.BlockSpec((1, OUT_DIM), lambda i: (0, 0)),
            pl.BlockSpec((1, OUT_DIM), lambda i: (0, 0)),
        ],
        out_specs=pl.BlockSpec((EDGE_TILE, HIDDEN), lambda i: (i, 0)),
        out_shape=jax.ShapeDtypeStruct((N_EDGES, HIDDEN), jnp.float32),
    )(z_bf, s1, s2, gamma_row, beta_row)


def _sc_scatter(messages, dst_idx):
    @functools.partial(
        pl.kernel,
        out_type=jax.ShapeDtypeStruct((NUM_CORES, N_NODES, HIDDEN),
                                      jnp.float32),
        mesh=_sc_mesh(),
        scratch_types=[
            pltpu.VMEM_SHARED((N_NODES, HIDDEN), jnp.float32),
            pltpu.VMEM((CHUNK, HIDDEN), jnp.float32),
            pltpu.VMEM((1, CHUNK), jnp.int32),
            pltpu.VMEM((ZROWS, HIDDEN), jnp.float32),
        ],
    )
    def k(m_hbm, di_hbm, out_hbm, acc_sh, m_v, idx_v, z_v):
        cid = jax.lax.axis_index("core")
        sid = jax.lax.axis_index("subcore")

        zvec = jnp.zeros((16,), jnp.float32)

        @pl.loop(0, ZROWS)
        def _(r):
            @pl.loop(0, HIDDEN, step=16)
            def _(c0):
                z_v[r, pl.ds(c0, 16)] = zvec

        my_rows = sid * ROWS_PER_SUBCORE

        @pl.loop(0, ROWS_PER_SUBCORE, step=ZROWS)
        def _(r0):
            pltpu.sync_copy(z_v, acc_sh.at[pl.ds(my_rows + r0, ZROWS)])

        plsc.subcore_barrier()

        base = (sid * NUM_CORES + cid) * EDGES_PER_WORKER

        @pl.loop(0, EDGES_PER_WORKER, step=CHUNK)
        def _(e0):
            e = base + e0
            pltpu.sync_copy(di_hbm.at[0, pl.ds(e, CHUNK)], idx_v.at[0])
            pltpu.sync_copy(m_hbm.at[pl.ds(e, CHUNK)], m_v)
            pltpu.sync_copy(m_v, acc_sh.at[idx_v.at[0]], add=True)

        plsc.subcore_barrier()
        pltpu.sync_copy(
            acc_sh.at[pl.ds(my_rows, ROWS_PER_SUBCORE)],
            out_hbm.at[cid, pl.ds(my_rows, ROWS_PER_SUBCORE)])

    return k(messages, dst_idx)


def _final(node_feats, partials):
    tile = 1000

    def body(nf_ref, p_ref, o_ref):
        o_ref[...] = jax.nn.softplus(nf_ref[...] + p_ref[0] + p_ref[1])

    return pl.pallas_call(
        body,
        grid=(N_NODES // tile,),
        in_specs=[
            pl.BlockSpec((tile, HIDDEN), lambda i: (i, 0)),
            pl.BlockSpec((NUM_CORES, tile, HIDDEN), lambda i: (0, i, 0)),
        ],
        out_specs=pl.BlockSpec((tile, HIDDEN), lambda i: (i, 0)),
        out_shape=jax.ShapeDtypeStruct((N_NODES, HIDDEN), jnp.float32),
    )(node_feats, partials)


def kernel(node_feats, edge_feats, edge_index, W, b, gamma, beta):
    edge_index = edge_index.astype(jnp.int32)
    src_idx = edge_index[0].reshape(1, N_EDGES)
    dst_idx = edge_index[1].reshape(1, N_EDGES)

    w12t = W[:, :2 * HIDDEN].T.astype(jnp.bfloat16)
    w3t = W[:, 2 * HIDDEN:].T.astype(jnp.bfloat16)
    b_row = b.reshape(1, OUT_DIM)
    gamma_row = gamma.reshape(1, OUT_DIM)
    beta_row = beta.reshape(1, OUT_DIM)

    src_rows, dst_rows = _sc_gather(node_feats, src_idx, dst_idx)
    z_bf, s1, s2 = _pass_a(src_rows, dst_rows, edge_feats, w12t, w3t, b_row)
    msgs = _pass_b(z_bf, s1, s2, gamma_row, beta_row)
    partials = _sc_scatter(msgs, dst_idx)
    return _final(node_feats, partials)


# speedup vs baseline: 2.7004x; 2.7004x over previous
import functools

import jax
import jax.numpy as jnp
from jax.experimental import pallas as pl
from jax.experimental.pallas import tpu as pltpu
from jax.experimental.pallas import tpu_sc as plsc

N_NODES = 10000
N_EDGES = 320000
HIDDEN = 128
EDGE_DIM = 16
OUT_DIM = 2 * HIDDEN
BN_EPS = 1e-5

NUM_CORES = 2
NUM_SUBCORES = 16
NUM_WORKERS = NUM_CORES * NUM_SUBCORES

GATHER_W = 128
EDGE_TILE = 1280
N_TILES = N_EDGES // EDGE_TILE

CHUNK = 128
N_CHUNKS = N_EDGES // CHUNK
CHUNKS_PER_WORKER = N_CHUNKS // NUM_WORKERS
REM_CHUNKS = N_CHUNKS - CHUNKS_PER_WORKER * NUM_WORKERS
N_NODES_PAD = 10240
ROWS_PER_SUBCORE = N_NODES_PAD // NUM_SUBCORES
ZROWS = 128


def _sc_mesh():
    return plsc.VectorSubcoreMesh(core_axis_name="core",
                                  subcore_axis_name="subcore")


def _sc_gather(node_feats, src_idx, dst_idx):
    out_t = jax.ShapeDtypeStruct((N_EDGES, HIDDEN), node_feats.dtype)

    @functools.partial(pl.kernel, out_type=(out_t, out_t), mesh=_sc_mesh())
    def k(nf_hbm, si_hbm, di_hbm, os_hbm, od_hbm):
        def body(si_v, di_v, os_v, od_v):
            pltpu.sync_copy(nf_hbm.at[si_v.at[0]], os_v)
            pltpu.sync_copy(nf_hbm.at[di_v.at[0]], od_v)

        pltpu.emit_pipeline(
            body,
            grid=(N_EDGES // GATHER_W,),
            in_specs=[
                pl.BlockSpec((1, GATHER_W), lambda i: (0, i)),
                pl.BlockSpec((1, GATHER_W), lambda i: (0, i)),
            ],
            out_specs=[
                pl.BlockSpec((GATHER_W, HIDDEN), lambda i: (i, 0)),
                pl.BlockSpec((GATHER_W, HIDDEN), lambda i: (i, 0)),
            ],
            core_axis_name=("core", "subcore"),
            dimension_semantics=(pltpu.PARALLEL,),
        )(si_hbm, di_hbm, os_hbm, od_hbm)

    return k(node_feats, src_idx, dst_idx)


def _pass_a(src_rows, dst_rows, edge_feats, w12t, w3t, b_row):
    def body(src_ref, dst_ref, ef_ref, w12_ref, w3_ref, b_ref,
             z_ref, s1_ref, s2_ref):
        x = jnp.concatenate([src_ref[...], dst_ref[...]], axis=1)
        z = jnp.dot(x.astype(jnp.bfloat16), w12_ref[...],
                    preferred_element_type=jnp.float32)
        z = z + jnp.dot(ef_ref[...].astype(jnp.bfloat16), w3_ref[...],
                        preferred_element_type=jnp.float32)
        z = z + b_ref[...]
        z_ref[...] = z.astype(jnp.bfloat16)

        @pl.when(pl.program_id(0) == 0)
        def _():
            s1_ref[...] = jnp.zeros_like(s1_ref)
            s2_ref[...] = jnp.zeros_like(s2_ref)

        s1_ref[...] += jnp.sum(z, axis=0, keepdims=True)
        s2_ref[...] += jnp.sum(z * z, axis=0, keepdims=True)

    return pl.pallas_call(
        body,
        grid=(N_TILES,),
        in_specs=[
            pl.BlockSpec((EDGE_TILE, HIDDEN), lambda i: (i, 0)),
            pl.BlockSpec((EDGE_TILE, HIDDEN), lambda i: (i, 0)),
            pl.BlockSpec((EDGE_TILE, EDGE_DIM), lambda i: (i, 0)),
            pl.BlockSpec((2 * HIDDEN, OUT_DIM), lambda i: (0, 0)),
            pl.BlockSpec((EDGE_DIM, OUT_DIM), lambda i: (0, 0)),
            pl.BlockSpec((1, OUT_DIM), lambda i: (0, 0)),
        ],
        out_specs=[
            pl.BlockSpec((EDGE_TILE, OUT_DIM), lambda i: (i, 0)),
            pl.BlockSpec((1, OUT_DIM), lambda i: (0, 0)),
            pl.BlockSpec((1, OUT_DIM), lambda i: (0, 0)),
        ],
        out_shape=[
            jax.ShapeDtypeStruct((N_EDGES, OUT_DIM), jnp.bfloat16),
            jax.ShapeDtypeStruct((1, OUT_DIM), jnp.float32),
            jax.ShapeDtypeStruct((1, OUT_DIM), jnp.float32),
        ],
    )(src_rows, dst_rows, edge_feats, w12t, w3t, b_row)


def _pass_b(z_bf, s1, s2, gamma_row, beta_row):
    def body(z_ref, s1_ref, s2_ref, g_ref, be_ref, m_ref):
        inv_n = jnp.float32(1.0 / N_EDGES)
        mean = s1_ref[...] * inv_n
        var = s2_ref[...] * inv_n - mean * mean
        scale = g_ref[...] * jax.lax.rsqrt(var + BN_EPS)
        shift = be_ref[...] - mean * scale
        zn = z_ref[...].astype(jnp.float32) * scale + shift
        gate = zn[:, :HIDDEN]
        msg = zn[:, HIDDEN:]
        m_ref[...] = jax.nn.sigmoid(gate) * jax.nn.softplus(msg)

    return pl.pallas_call(
        body,
        grid=(N_TILES,),
        in_specs=[
            pl.BlockSpec((EDGE_TILE, OUT_DIM), lambda i: (i, 0)),
            pl.BlockSpec((1, OUT_DIM), lambda i: (0, 0)),
            pl.BlockSpec((1, OUT_DIM), lambda i: (0, 0)),
            pl.BlockSpec((1, OUT_DIM), lambda i: (0, 0)),
            pl.BlockSpec((1, OUT_DIM), lambda i: (0, 0)),
        ],
        out_specs=pl.BlockSpec((EDGE_TILE, HIDDEN), lambda i: (i, 0)),
        out_shape=jax.ShapeDtypeStruct((N_EDGES, HIDDEN), jnp.float32),
    )(z_bf, s1, s2, gamma_row, beta_row)


def _sc_scatter(messages, dst_idx):
    @functools.partial(
        pl.kernel,
        out_type=jax.ShapeDtypeStruct((NUM_CORES, N_NODES_PAD, HIDDEN),
                                      jnp.float32),
        mesh=_sc_mesh(),
        scratch_types=[
            pltpu.VMEM_SHARED((N_NODES_PAD, HIDDEN), jnp.float32),
            pltpu.VMEM((CHUNK, HIDDEN), jnp.float32),
            pltpu.VMEM((1, CHUNK), jnp.int32),
            pltpu.VMEM((ZROWS, HIDDEN), jnp.float32),
        ],
    )
    def k(m_hbm, di_hbm, out_hbm, acc_sh, m_v, idx_v, z_v):
        cid = jax.lax.axis_index("core")
        sid = jax.lax.axis_index("subcore")

        zvec = jnp.zeros((16,), jnp.float32)

        @pl.loop(0, ZROWS)
        def _(r):
            @pl.loop(0, HIDDEN, step=16)
            def _(c0):
                z_v[r, pl.ds(c0, 16)] = zvec

        my_rows = sid * ROWS_PER_SUBCORE

        @pl.loop(0, ROWS_PER_SUBCORE, step=ZROWS)
        def _(r0):
            pltpu.sync_copy(z_v, acc_sh.at[pl.ds(my_rows + r0, ZROWS)])

        plsc.subcore_barrier()

        wid = sid * NUM_CORES + cid

        def do_chunk(c):
            pltpu.sync_copy(di_hbm.at[c], idx_v.at[0])
            pltpu.sync_copy(m_hbm.at[pl.ds(c * CHUNK, CHUNK)], m_v)
            pltpu.sync_copy(m_v, acc_sh.at[idx_v.at[0]], add=True)

        @pl.loop(0, CHUNKS_PER_WORKER)
        def _(j):
            do_chunk(wid * CHUNKS_PER_WORKER + j)

        @pl.when(wid < REM_CHUNKS)
        def _():
            do_chunk(NUM_WORKERS * CHUNKS_PER_WORKER + wid)

        plsc.subcore_barrier()
        pltpu.sync_copy(
            acc_sh.at[pl.ds(my_rows, ROWS_PER_SUBCORE)],
            out_hbm.at[cid, pl.ds(my_rows, ROWS_PER_SUBCORE)])

    return k(messages, dst_idx)


def _final(node_feats, partials):
    tile = 1000

    def body(nf_ref, p_ref, o_ref):
        o_ref[...] = jax.nn.softplus(nf_ref[...] + p_ref[0] + p_ref[1])

    return pl.pallas_call(
        body,
        grid=(N_NODES // tile,),
        in_specs=[
            pl.BlockSpec((tile, HIDDEN), lambda i: (i, 0)),
            pl.BlockSpec((NUM_CORES, tile, HIDDEN), lambda i: (0, i, 0)),
        ],
        out_specs=pl.BlockSpec((tile, HIDDEN), lambda i: (i, 0)),
        out_shape=jax.ShapeDtypeStruct((N_NODES, HIDDEN), jnp.float32),
    )(node_feats, partials)


def kernel(node_feats, edge_feats, edge_index, W, b, gamma, beta):
    edge_index = edge_index.astype(jnp.int32)
    src_idx = edge_index[0].reshape(1, N_EDGES)
    dst_idx = edge_index[1].reshape(1, N_EDGES)

    w12t = W[:, :2 * HIDDEN].T.astype(jnp.bfloat16)
    w3t = W[:, 2 * HIDDEN:].T.astype(jnp.bfloat16)
    b_row = b.reshape(1, OUT_DIM)
    gamma_row = gamma.reshape(1, OUT_DIM)
    beta_row = beta.reshape(1, OUT_DIM)

    src_rows, dst_rows = _sc_gather(node_feats, src_idx, dst_idx)
    z_bf, s1, s2 = _pass_a(src_rows, dst_rows, edge_feats, w12t, w3t, b_row)
    msgs = _pass_b(z_bf, s1, s2, gamma_row, beta_row)
    partials = _sc_scatter(msgs, dst_idx.reshape(N_CHUNKS, CHUNK))
    return _final(node_feats, partials)
